# Initial kernel scaffold; baseline (speedup 1.0000x reference)
import jax
import jax.numpy as jnp
from jax.experimental import pallas as pl


def kernel(gconv1_w, gconv1_b, lupool1_p, lupool1_w, gconv2_w, gconv2_b, lupool2_p, lupool2_w, gconv3_w, gconv3_b, lupool3_p, lupool3_w, cat_a_w, cat_a_b, cat_b_w, cat_b_b, a_raw, x_feat, mask, pad_dmap):
    raise NotImplementedError("write your pallas kernel here")


# speedup vs baseline: 1.1353x; 1.1353x over previous
import functools

import jax
import jax.numpy as jnp
from jax.experimental import pallas as pl
from jax.experimental.pallas import tpu as pltpu

_NEG_BIG = -1e30


def _outer(u, v):
    return jax.lax.dot_general(u, v, (((1,), (1,)), ((), ())),
                               preferred_element_type=jnp.float32)


def _fused_kernel(a_ref, x_ref, m_ref,
                  w1_ref, b1_ref, p1_ref, wp1_ref,
                  w2_ref, b2_ref, p2_ref, wp2_ref,
                  w3_ref, b3_ref, p3_ref, wp3_ref,
                  wa_ref, ba_ref, wb_ref, bb_ref,
                  o_ref, *, n):
    idx_r = jax.lax.broadcasted_iota(jnp.int32, (n, n), 0)
    idx_c = jax.lax.broadcasted_iota(jnp.int32, (n, n), 1)
    ones = jnp.ones((n, 1), jnp.float32)

    def stage(a_cur, x, m, w_ref, b_ref, p_ref, wp_ref, k_keep, need_pool):
        am = a_cur * _outer(m, m)
        deg = jnp.sum(am, axis=1, keepdims=True)
        dinv = jnp.where(deg > 0.0,
                         jax.lax.rsqrt(jnp.maximum(deg, 1e-12)), 0.0)
        a_norm = am * _outer(dinv, dinv)

        h = jnp.dot(a_norm,
                    jnp.dot(x, w_ref[...], preferred_element_type=jnp.float32),
                    preferred_element_type=jnp.float32) + b_ref[...]
        h = jnp.maximum(h, 0.0) * m

        s = jnp.dot(h, p_ref[...], preferred_element_type=jnp.float32)
        s = jnp.where(m > 0.0, s, _NEG_BIG)

        s_cols = _outer(ones, s)
        beats = jnp.logical_or(
            s > s_cols, jnp.logical_and(s == s_cols, idx_r < idx_c))
        wins = jnp.sum(beats.astype(jnp.float32), axis=1, keepdims=True)
        keep = jnp.where((n - 1.0) - wins < k_keep, m, 0.0)

        x_new = jnp.dot(h * jnp.tanh(s) * keep, wp_ref[...],
                        preferred_element_type=jnp.float32) * keep

        a_next = a_cur * _outer(keep, keep) if need_pool else None

        cnt = jnp.sum(keep, axis=0, keepdims=True)
        avg = jnp.sum(x_new, axis=0, keepdims=True) / jnp.maximum(cnt, 1.0)
        mx = jnp.max(jnp.where(keep > 0.0, x_new, _NEG_BIG),
                     axis=0, keepdims=True)
        mx = jnp.where(cnt > 0.0, mx, 0.0)
        ro = jnp.concatenate([avg, mx], axis=1)
        return a_next, x_new, keep, ro

    a1, x1, m1, r1 = stage(a_ref[0], x_ref[0], m_ref[0],
                           w1_ref, b1_ref, p1_ref, wp1_ref, n // 2, True)
    a2, x2, m2, r2 = stage(a1, x1, m1,
                           w2_ref, b2_ref, p2_ref, wp2_ref, n // 4, True)
    _, _, _, r3 = stage(a2, x2, m2,
                        w3_ref, b3_ref, p3_ref, wp3_ref, n // 8, False)

    out = (jnp.dot(r1, wa_ref[...], preferred_element_type=jnp.float32)
           + ba_ref[...]
           + jnp.dot(r2, wb_ref[...], preferred_element_type=jnp.float32)
           + bb_ref[...]
           + r3)
    o_ref[0] = out


def kernel(gconv1_w, gconv1_b, lupool1_p, lupool1_w,
           gconv2_w, gconv2_b, lupool2_p, lupool2_w,
           gconv3_w, gconv3_b, lupool3_p, lupool3_w,
           cat_a_w, cat_a_b, cat_b_w, cat_b_b,
           a_raw, x_feat, mask, pad_dmap):
    del pad_dmap
    B, N, Fin = x_feat.shape
    E = cat_a_w.shape[1]

    const = lambda shape: pl.BlockSpec(shape, lambda i: (0,) * len(shape))
    weights = [
        gconv1_w, gconv1_b.reshape(1, -1), lupool1_p, lupool1_w,
        gconv2_w, gconv2_b.reshape(1, -1), lupool2_p, lupool2_w,
        gconv3_w, gconv3_b.reshape(1, -1), lupool3_p, lupool3_w,
        cat_a_w, cat_a_b.reshape(1, -1), cat_b_w, cat_b_b.reshape(1, -1),
    ]

    out = pl.pallas_call(
        functools.partial(_fused_kernel, n=N),
        out_shape=jax.ShapeDtypeStruct((B, 1, E), jnp.float32),
        grid=(B,),
        in_specs=[
            pl.BlockSpec((1, N, N), lambda i: (i, 0, 0)),
            pl.BlockSpec((1, N, Fin), lambda i: (i, 0, 0)),
            pl.BlockSpec((1, N, 1), lambda i: (i, 0, 0)),
        ] + [const(w.shape) for w in weights],
        out_specs=pl.BlockSpec((1, 1, E), lambda i: (i, 0, 0)),
        compiler_params=pltpu.CompilerParams(
            dimension_semantics=("parallel",)),
    )(a_raw, x_feat, mask, *weights)
    return out.reshape(B, E)


# speedup vs baseline: 1.3831x; 1.2183x over previous
import functools

import jax
import jax.numpy as jnp
from jax.experimental import pallas as pl
from jax.experimental.pallas import tpu as pltpu

_NEG_BIG = -1e30


def _outer(u, v):
    return jax.lax.dot_general(u, v, (((1,), (1,)), ((), ())),
                               preferred_element_type=jnp.float32)


def _fused_kernel(a_ref, x_ref, m_ref,
                  w1_ref, b1_ref, p1_ref, wp1_ref,
                  w2_ref, b2_ref, p2_ref, wp2_ref,
                  w3_ref, b3_ref, p3_ref, wp3_ref,
                  wa_ref, ba_ref, wb_ref, bb_ref,
                  o_ref, *, n):
    idx_r = jax.lax.broadcasted_iota(jnp.int32, (n, n), 0)
    idx_c = jax.lax.broadcasted_iota(jnp.int32, (n, n), 1)
    ones = jnp.ones((n, 1), jnp.float32)

    a_raw = a_ref[0]
    tie = idx_r < idx_c

    def stage(x, m, w_ref, b_ref, p_ref, wp_ref, k_keep):
        deg = m * jnp.dot(a_raw, m, preferred_element_type=jnp.float32)
        dinv = jnp.where(deg > 0.0,
                         jax.lax.rsqrt(jnp.maximum(deg, 1e-12)), 0.0)
        a_norm = a_raw * _outer(dinv, dinv)

        h = jnp.dot(a_norm,
                    jnp.dot(x, w_ref[...], preferred_element_type=jnp.float32),
                    preferred_element_type=jnp.float32) + b_ref[...]
        h = jnp.maximum(h, 0.0) * m

        s = jnp.dot(h, p_ref[...], preferred_element_type=jnp.float32)
        s = jnp.where(m > 0.0, s, _NEG_BIG)

        s_cols = _outer(ones, s)
        beats = jnp.logical_or(
            s > s_cols, jnp.logical_and(s == s_cols, tie))
        wins = jnp.dot(beats.astype(jnp.float32), ones,
                       preferred_element_type=jnp.float32)
        keep = jnp.where((n - 1.0) - wins < k_keep, m, 0.0)

        x_new = jnp.dot(h * jnp.tanh(s) * keep, wp_ref[...],
                        preferred_element_type=jnp.float32) * keep

        cnt = jnp.sum(keep, axis=0, keepdims=True)
        avg = jnp.sum(x_new, axis=0, keepdims=True) / jnp.maximum(cnt, 1.0)
        mx = jnp.max(jnp.where(keep > 0.0, x_new, _NEG_BIG),
                     axis=0, keepdims=True)
        mx = jnp.where(cnt > 0.0, mx, 0.0)
        ro = jnp.concatenate([avg, mx], axis=1)
        return x_new, keep, ro

    x1, m1, r1 = stage(x_ref[0], m_ref[0],
                       w1_ref, b1_ref, p1_ref, wp1_ref, n // 2)
    x2, m2, r2 = stage(x1, m1, w2_ref, b2_ref, p2_ref, wp2_ref, n // 4)
    _, _, r3 = stage(x2, m2, w3_ref, b3_ref, p3_ref, wp3_ref, n // 8)

    out = (jnp.dot(r1, wa_ref[...], preferred_element_type=jnp.float32)
           + ba_ref[...]
           + jnp.dot(r2, wb_ref[...], preferred_element_type=jnp.float32)
           + bb_ref[...]
           + r3)
    o_ref[0] = out


def kernel(gconv1_w, gconv1_b, lupool1_p, lupool1_w,
           gconv2_w, gconv2_b, lupool2_p, lupool2_w,
           gconv3_w, gconv3_b, lupool3_p, lupool3_w,
           cat_a_w, cat_a_b, cat_b_w, cat_b_b,
           a_raw, x_feat, mask, pad_dmap):
    del pad_dmap
    B, N, Fin = x_feat.shape
    E = cat_a_w.shape[1]

    const = lambda shape: pl.BlockSpec(shape, lambda i: (0,) * len(shape))
    weights = [
        gconv1_w, gconv1_b.reshape(1, -1), lupool1_p, lupool1_w,
        gconv2_w, gconv2_b.reshape(1, -1), lupool2_p, lupool2_w,
        gconv3_w, gconv3_b.reshape(1, -1), lupool3_p, lupool3_w,
        cat_a_w, cat_a_b.reshape(1, -1), cat_b_w, cat_b_b.reshape(1, -1),
    ]

    out = pl.pallas_call(
        functools.partial(_fused_kernel, n=N),
        out_shape=jax.ShapeDtypeStruct((B, 1, E), jnp.float32),
        grid=(B,),
        in_specs=[
            pl.BlockSpec((1, N, N), lambda i: (i, 0, 0)),
            pl.BlockSpec((1, N, Fin), lambda i: (i, 0, 0)),
            pl.BlockSpec((1, N, 1), lambda i: (i, 0, 0)),
        ] + [const(w.shape) for w in weights],
        out_specs=pl.BlockSpec((1, 1, E), lambda i: (i, 0, 0)),
        compiler_params=pltpu.CompilerParams(
            dimension_semantics=("parallel",)),
    )(a_raw, x_feat, mask, *weights)
    return out.reshape(B, E)


# speedup vs baseline: 1.3835x; 1.0003x over previous
import functools

import jax
import jax.numpy as jnp
from jax.experimental import pallas as pl
from jax.experimental.pallas import tpu as pltpu

_NEG_BIG = -1e30


def _outer(u, v):
    return jax.lax.dot_general(u, v, (((1,), (1,)), ((), ())),
                               preferred_element_type=jnp.float32)


def _fused_kernel(a_ref, x_ref, m_ref,
                  w1_ref, b1_ref, p1_ref, wp1_ref,
                  w2_ref, b2_ref, p2_ref, wp2_ref,
                  w3_ref, b3_ref, p3_ref, wp3_ref,
                  wa_ref, ba_ref, wb_ref, bb_ref,
                  o_ref, *, n):
    idx_r = jax.lax.broadcasted_iota(jnp.int32, (n, n), 0)
    idx_c = jax.lax.broadcasted_iota(jnp.int32, (n, n), 1)
    ones = jnp.ones((n, 1), jnp.float32)
    tie = idx_r < idx_c
    npair = a_ref.shape[0]

    def stage(a_raw, a_bf, x, m, w_ref, b_ref, p_ref, wp_ref, k_keep):
        deg = m * jnp.dot(a_bf, m.astype(jnp.bfloat16),
                          preferred_element_type=jnp.float32)
        dinv = jnp.where(deg > 0.0,
                         jax.lax.rsqrt(jnp.maximum(deg, 1e-12)), 0.0)
        a_norm = a_raw * _outer(dinv, dinv)

        h = jnp.dot(a_norm,
                    jnp.dot(x, w_ref[...], preferred_element_type=jnp.float32),
                    preferred_element_type=jnp.float32) + b_ref[...]
        h = jnp.maximum(h, 0.0) * m

        s = jnp.dot(h, p_ref[...], preferred_element_type=jnp.float32)
        s = jnp.where(m > 0.0, s, _NEG_BIG)

        s_cols = _outer(ones, s)
        beats = jnp.logical_or(
            s > s_cols, jnp.logical_and(s == s_cols, tie))
        wins = jnp.dot(beats.astype(jnp.bfloat16), ones.astype(jnp.bfloat16),
                       preferred_element_type=jnp.float32)
        keep = jnp.where((n - 1.0) - wins < k_keep, m, 0.0)

        x_new = jnp.dot(h * jnp.tanh(s) * keep, wp_ref[...],
                        preferred_element_type=jnp.float32) * keep

        cnt = jnp.sum(keep, axis=0, keepdims=True)
        avg = jnp.sum(x_new, axis=0, keepdims=True) / jnp.maximum(cnt, 1.0)
        mx = jnp.max(jnp.where(keep > 0.0, x_new, _NEG_BIG),
                     axis=0, keepdims=True)
        mx = jnp.where(cnt > 0.0, mx, 0.0)
        ro = jnp.concatenate([avg, mx], axis=1)
        return x_new, keep, ro

    for g in range(npair):
        a_raw = a_ref[g]
        a_bf = a_raw.astype(jnp.bfloat16)
        x1, m1, r1 = stage(a_raw, a_bf, x_ref[g], m_ref[g],
                           w1_ref, b1_ref, p1_ref, wp1_ref, n // 2)
        x2, m2, r2 = stage(a_raw, a_bf, x1, m1,
                           w2_ref, b2_ref, p2_ref, wp2_ref, n // 4)
        _, _, r3 = stage(a_raw, a_bf, x2, m2,
                         w3_ref, b3_ref, p3_ref, wp3_ref, n // 8)

        out = (jnp.dot(r1, wa_ref[...], preferred_element_type=jnp.float32)
               + ba_ref[...]
               + jnp.dot(r2, wb_ref[...], preferred_element_type=jnp.float32)
               + bb_ref[...]
               + r3)
        o_ref[g] = out


def kernel(gconv1_w, gconv1_b, lupool1_p, lupool1_w,
           gconv2_w, gconv2_b, lupool2_p, lupool2_w,
           gconv3_w, gconv3_b, lupool3_p, lupool3_w,
           cat_a_w, cat_a_b, cat_b_w, cat_b_b,
           a_raw, x_feat, mask, pad_dmap):
    del pad_dmap
    B, N, Fin = x_feat.shape
    E = cat_a_w.shape[1]

    const = lambda shape: pl.BlockSpec(shape, lambda i: (0,) * len(shape))
    weights = [
        gconv1_w, gconv1_b.reshape(1, -1), lupool1_p, lupool1_w,
        gconv2_w, gconv2_b.reshape(1, -1), lupool2_p, lupool2_w,
        gconv3_w, gconv3_b.reshape(1, -1), lupool3_p, lupool3_w,
        cat_a_w, cat_a_b.reshape(1, -1), cat_b_w, cat_b_b.reshape(1, -1),
    ]

    G = 2 if B % 2 == 0 else 1
    out = pl.pallas_call(
        functools.partial(_fused_kernel, n=N),
        out_shape=jax.ShapeDtypeStruct((B, 1, E), jnp.float32),
        grid=(B // G,),
        in_specs=[
            pl.BlockSpec((G, N, N), lambda i: (i, 0, 0)),
            pl.BlockSpec((G, N, Fin), lambda i: (i, 0, 0)),
            pl.BlockSpec((G, N, 1), lambda i: (i, 0, 0)),
        ] + [const(w.shape) for w in weights],
        out_specs=pl.BlockSpec((G, 1, E), lambda i: (i, 0, 0)),
        compiler_params=pltpu.CompilerParams(
            dimension_semantics=("parallel",)),
    )(a_raw, x_feat, mask, *weights)
    return out.reshape(B, E)


# speedup vs baseline: 1.9081x; 1.3792x over previous
import functools

import jax
import jax.numpy as jnp
from jax.experimental import pallas as pl
from jax.experimental.pallas import tpu as pltpu

_NEG_BIG = -1e30


def _outer(u, v):
    return jax.lax.dot_general(u, v, (((1,), (1,)), ((), ())),
                               preferred_element_type=jnp.float32)


def _fused_kernel(a_ref, x_ref, m_ref,
                  w1_ref, b1_ref, p1_ref, wp1_ref,
                  w2_ref, b2_ref, p2_ref, wp2_ref,
                  w3_ref, b3_ref, p3_ref, wp3_ref,
                  wa_ref, ba_ref, wb_ref, bb_ref,
                  o_ref, *, n):
    idx_r = jax.lax.broadcasted_iota(jnp.int32, (n, n), 0)
    idx_c = jax.lax.broadcasted_iota(jnp.int32, (n, n), 1)
    ones = jnp.ones((n, 1), jnp.float32)
    tie = idx_r < idx_c
    npair = a_ref.shape[0]

    gs = range(npair)

    def each(f):
        return [f(g) for g in gs]

    def stage(a_raw, a_bf, x, m, w_ref, b_ref, p_ref, wp_ref, k_keep):
        deg = each(lambda g: m[g] * jnp.dot(
            a_bf[g], m[g].astype(jnp.bfloat16),
            preferred_element_type=jnp.float32))
        dinv = each(lambda g: jnp.where(
            deg[g] > 0.0, jax.lax.rsqrt(jnp.maximum(deg[g], 1e-12)), 0.0))
        a_norm = each(lambda g: a_raw[g] * _outer(dinv[g], dinv[g]))

        xw = each(lambda g: jnp.dot(x[g], w_ref[...],
                                    preferred_element_type=jnp.float32))
        h = each(lambda g: jnp.dot(a_norm[g], xw[g],
                                   preferred_element_type=jnp.float32)
                 + b_ref[...])
        h = each(lambda g: jnp.maximum(h[g], 0.0) * m[g])

        s = each(lambda g: jnp.dot(h[g], p_ref[...],
                                   preferred_element_type=jnp.float32))
        s = each(lambda g: jnp.where(m[g] > 0.0, s[g], _NEG_BIG))

        s_cols = each(lambda g: _outer(ones, s[g]))
        beats = each(lambda g: jnp.logical_or(
            s[g] > s_cols[g], jnp.logical_and(s[g] == s_cols[g], tie)))
        wins = each(lambda g: jnp.dot(
            beats[g].astype(jnp.bfloat16), ones.astype(jnp.bfloat16),
            preferred_element_type=jnp.float32))
        keep = each(lambda g: jnp.where((n - 1.0) - wins[g] < k_keep,
                                        m[g], 0.0))

        x_new = each(lambda g: jnp.dot(
            h[g] * jnp.tanh(s[g]) * keep[g], wp_ref[...],
            preferred_element_type=jnp.float32) * keep[g])

        cnt = each(lambda g: jnp.sum(keep[g], axis=0, keepdims=True))
        avg = each(lambda g: jnp.sum(x_new[g], axis=0, keepdims=True)
                   / jnp.maximum(cnt[g], 1.0))
        mx = each(lambda g: jnp.max(
            jnp.where(keep[g] > 0.0, x_new[g], _NEG_BIG),
            axis=0, keepdims=True))
        mx = each(lambda g: jnp.where(cnt[g] > 0.0, mx[g], 0.0))
        ro = each(lambda g: jnp.concatenate([avg[g], mx[g]], axis=1))
        return x_new, keep, ro

    a_raw = each(lambda g: a_ref[g])
    a_bf = each(lambda g: a_raw[g].astype(jnp.bfloat16))
    x0 = each(lambda g: x_ref[g])
    m0 = each(lambda g: m_ref[g])
    x1, m1, r1 = stage(a_raw, a_bf, x0, m0,
                       w1_ref, b1_ref, p1_ref, wp1_ref, n // 2)
    x2, m2, r2 = stage(a_raw, a_bf, x1, m1,
                       w2_ref, b2_ref, p2_ref, wp2_ref, n // 4)
    _, _, r3 = stage(a_raw, a_bf, x2, m2,
                     w3_ref, b3_ref, p3_ref, wp3_ref, n // 8)

    out = each(lambda g: jnp.dot(r1[g], wa_ref[...],
                                 preferred_element_type=jnp.float32)
               + ba_ref[...]
               + jnp.dot(r2[g], wb_ref[...],
                         preferred_element_type=jnp.float32)
               + bb_ref[...]
               + r3[g])
    for g in gs:
        o_ref[g] = out[g]


def kernel(gconv1_w, gconv1_b, lupool1_p, lupool1_w,
           gconv2_w, gconv2_b, lupool2_p, lupool2_w,
           gconv3_w, gconv3_b, lupool3_p, lupool3_w,
           cat_a_w, cat_a_b, cat_b_w, cat_b_b,
           a_raw, x_feat, mask, pad_dmap):
    del pad_dmap
    B, N, Fin = x_feat.shape
    E = cat_a_w.shape[1]

    const = lambda shape: pl.BlockSpec(shape, lambda i: (0,) * len(shape))
    weights = [
        gconv1_w, gconv1_b.reshape(1, -1), lupool1_p, lupool1_w,
        gconv2_w, gconv2_b.reshape(1, -1), lupool2_p, lupool2_w,
        gconv3_w, gconv3_b.reshape(1, -1), lupool3_p, lupool3_w,
        cat_a_w, cat_a_b.reshape(1, -1), cat_b_w, cat_b_b.reshape(1, -1),
    ]

    G = 2 if B % 2 == 0 else 1
    out = pl.pallas_call(
        functools.partial(_fused_kernel, n=N),
        out_shape=jax.ShapeDtypeStruct((B, 1, E), jnp.float32),
        grid=(B // G,),
        in_specs=[
            pl.BlockSpec((G, N, N), lambda i: (i, 0, 0)),
            pl.BlockSpec((G, N, Fin), lambda i: (i, 0, 0)),
            pl.BlockSpec((G, N, 1), lambda i: (i, 0, 0)),
        ] + [const(w.shape) for w in weights],
        out_specs=pl.BlockSpec((G, 1, E), lambda i: (i, 0, 0)),
        compiler_params=pltpu.CompilerParams(
            dimension_semantics=("parallel",)),
    )(a_raw, x_feat, mask, *weights)
    return out.reshape(B, E)


# speedup vs baseline: 2.0592x; 1.0792x over previous
import functools

import jax
import jax.numpy as jnp
from jax.experimental import pallas as pl
from jax.experimental.pallas import tpu as pltpu

_NEG_BIG = -1e30


def _outer(u, v):
    return jax.lax.dot_general(u, v, (((1,), (1,)), ((), ())),
                               preferred_element_type=jnp.float32)


def _fused_kernel(a_ref, x_ref, m_ref,
                  w1_ref, b1_ref, p1_ref, wp1_ref,
                  w2_ref, b2_ref, p2_ref, wp2_ref,
                  w3_ref, b3_ref, p3_ref, wp3_ref,
                  wa_ref, ba_ref, wb_ref, bb_ref,
                  o_ref, *, n):
    idx_r = jax.lax.broadcasted_iota(jnp.int32, (n, n), 0)
    idx_c = jax.lax.broadcasted_iota(jnp.int32, (n, n), 1)
    ones = jnp.ones((n, 1), jnp.float32)
    tie = idx_r < idx_c
    npair = a_ref.shape[0]

    gs = range(npair)

    def each(f):
        return [f(g) for g in gs]

    def stage(a_raw, a_bf, x, m, w_ref, b_ref, p_ref, wp_ref, k_keep):
        deg = each(lambda g: m[g] * jnp.dot(
            a_bf[g], m[g].astype(jnp.bfloat16),
            preferred_element_type=jnp.float32))
        dinv = each(lambda g: jnp.where(
            deg[g] > 0.0, jax.lax.rsqrt(jnp.maximum(deg[g], 1e-12)), 0.0))
        a_norm = each(lambda g: a_raw[g] * _outer(dinv[g], dinv[g]))

        xw = each(lambda g: jnp.dot(x[g], w_ref[...],
                                    preferred_element_type=jnp.float32))
        h = each(lambda g: jnp.dot(a_norm[g], xw[g],
                                   preferred_element_type=jnp.float32)
                 + b_ref[...])
        h = each(lambda g: jnp.maximum(h[g], 0.0) * m[g])

        s = each(lambda g: jnp.dot(h[g], p_ref[...],
                                   preferred_element_type=jnp.float32))
        s = each(lambda g: jnp.where(m[g] > 0.0, s[g], _NEG_BIG))

        s_cols = each(lambda g: _outer(ones, s[g]))
        beats = each(lambda g: jnp.logical_or(
            s[g] > s_cols[g], jnp.logical_and(s[g] == s_cols[g], tie)))
        wins = each(lambda g: jnp.dot(
            beats[g].astype(jnp.bfloat16), ones.astype(jnp.bfloat16),
            preferred_element_type=jnp.float32))
        keep = each(lambda g: jnp.where((n - 1.0) - wins[g] < k_keep,
                                        m[g], 0.0))

        x_new = each(lambda g: jnp.dot(
            h[g] * jnp.tanh(s[g]) * keep[g], wp_ref[...],
            preferred_element_type=jnp.float32) * keep[g])

        cnt = each(lambda g: jnp.sum(keep[g], axis=0, keepdims=True))
        avg = each(lambda g: jnp.sum(x_new[g], axis=0, keepdims=True)
                   / jnp.maximum(cnt[g], 1.0))
        mx = each(lambda g: jnp.max(
            jnp.where(keep[g] > 0.0, x_new[g], _NEG_BIG),
            axis=0, keepdims=True))
        mx = each(lambda g: jnp.where(cnt[g] > 0.0, mx[g], 0.0))
        ro = each(lambda g: jnp.concatenate([avg[g], mx[g]], axis=1))
        return x_new, keep, ro

    a_raw = each(lambda g: a_ref[g])
    a_bf = each(lambda g: a_raw[g].astype(jnp.bfloat16))
    x0 = each(lambda g: x_ref[g])
    m0 = each(lambda g: m_ref[g])
    x1, m1, r1 = stage(a_raw, a_bf, x0, m0,
                       w1_ref, b1_ref, p1_ref, wp1_ref, n // 2)
    x2, m2, r2 = stage(a_raw, a_bf, x1, m1,
                       w2_ref, b2_ref, p2_ref, wp2_ref, n // 4)
    _, _, r3 = stage(a_raw, a_bf, x2, m2,
                     w3_ref, b3_ref, p3_ref, wp3_ref, n // 8)

    out = each(lambda g: jnp.dot(r1[g], wa_ref[...],
                                 preferred_element_type=jnp.float32)
               + ba_ref[...]
               + jnp.dot(r2[g], wb_ref[...],
                         preferred_element_type=jnp.float32)
               + bb_ref[...]
               + r3[g])
    for g in gs:
        o_ref[g] = out[g]


def kernel(gconv1_w, gconv1_b, lupool1_p, lupool1_w,
           gconv2_w, gconv2_b, lupool2_p, lupool2_w,
           gconv3_w, gconv3_b, lupool3_p, lupool3_w,
           cat_a_w, cat_a_b, cat_b_w, cat_b_b,
           a_raw, x_feat, mask, pad_dmap):
    del pad_dmap
    B, N, Fin = x_feat.shape
    E = cat_a_w.shape[1]

    const = lambda shape: pl.BlockSpec(shape, lambda i: (0,) * len(shape))
    weights = [
        gconv1_w, gconv1_b.reshape(1, -1), lupool1_p, lupool1_w,
        gconv2_w, gconv2_b.reshape(1, -1), lupool2_p, lupool2_w,
        gconv3_w, gconv3_b.reshape(1, -1), lupool3_p, lupool3_w,
        cat_a_w, cat_a_b.reshape(1, -1), cat_b_w, cat_b_b.reshape(1, -1),
    ]

    G = 4 if B % 4 == 0 else (2 if B % 2 == 0 else 1)
    out = pl.pallas_call(
        functools.partial(_fused_kernel, n=N),
        out_shape=jax.ShapeDtypeStruct((B, 1, E), jnp.float32),
        grid=(B // G,),
        in_specs=[
            pl.BlockSpec((G, N, N), lambda i: (i, 0, 0)),
            pl.BlockSpec((G, N, Fin), lambda i: (i, 0, 0)),
            pl.BlockSpec((G, N, 1), lambda i: (i, 0, 0)),
        ] + [const(w.shape) for w in weights],
        out_specs=pl.BlockSpec((G, 1, E), lambda i: (i, 0, 0)),
        compiler_params=pltpu.CompilerParams(
            dimension_semantics=("parallel",)),
    )(a_raw, x_feat, mask, *weights)
    return out.reshape(B, E)


# speedup vs baseline: 2.3117x; 1.1226x over previous
import functools

import jax
import jax.numpy as jnp
from jax.experimental import pallas as pl
from jax.experimental.pallas import tpu as pltpu

_NEG_BIG = -1e30


def _outer(u, v):
    return jax.lax.dot_general(u, v, (((1,), (1,)), ((), ())),
                               preferred_element_type=jnp.float32)


def _fused_kernel(a_ref, x_ref, m_ref,
                  w1_ref, b1_ref, p1_ref, wp1_ref,
                  w2_ref, b2_ref, p2_ref, wp2_ref,
                  w3_ref, b3_ref, p3_ref, wp3_ref,
                  wa_ref, ba_ref, wb_ref, bb_ref,
                  o_ref, *, n):
    idx_r = jax.lax.broadcasted_iota(jnp.int32, (n, n), 0)
    idx_c = jax.lax.broadcasted_iota(jnp.int32, (n, n), 1)
    ones = jnp.ones((n, 1), jnp.float32)
    tie = idx_r < idx_c
    npair = a_ref.shape[0]

    gs = range(npair)

    def each(f):
        return [f(g) for g in gs]

    def stage(a_raw, a_bf, x, m, w_ref, b_ref, p_ref, wp_ref, k_keep):
        deg = each(lambda g: m[g] * jnp.dot(
            a_bf[g], m[g].astype(jnp.bfloat16),
            preferred_element_type=jnp.float32))
        dinv = each(lambda g: jnp.where(
            deg[g] > 0.0, jax.lax.rsqrt(jnp.maximum(deg[g], 1e-12)), 0.0))
        dinv_t = each(lambda g: jax.lax.transpose(dinv[g], (1, 0)))
        a_norm = each(lambda g: (a_raw[g] * dinv[g]) * dinv_t[g])

        xw = each(lambda g: jnp.dot(x[g], w_ref[...],
                                    preferred_element_type=jnp.float32))
        h = each(lambda g: jnp.dot(a_norm[g], xw[g],
                                   preferred_element_type=jnp.float32)
                 + b_ref[...])
        h = each(lambda g: jnp.maximum(h[g], 0.0) * m[g])

        s = each(lambda g: jnp.dot(h[g], p_ref[...],
                                   preferred_element_type=jnp.float32))
        s = each(lambda g: jnp.where(m[g] > 0.0, s[g], _NEG_BIG))

        s_cols = each(lambda g: jax.lax.transpose(s[g], (1, 0)))
        beats = each(lambda g: jnp.logical_or(
            s[g] > s_cols[g], jnp.logical_and(s[g] == s_cols[g], tie)))
        wins = each(lambda g: jnp.dot(
            beats[g].astype(jnp.bfloat16), ones.astype(jnp.bfloat16),
            preferred_element_type=jnp.float32))
        keep = each(lambda g: jnp.where((n - 1.0) - wins[g] < k_keep,
                                        m[g], 0.0))

        x_new = each(lambda g: jnp.dot(
            h[g] * jnp.tanh(s[g]) * keep[g], wp_ref[...],
            preferred_element_type=jnp.float32) * keep[g])

        cnt = each(lambda g: jnp.sum(keep[g], axis=0, keepdims=True))
        avg = each(lambda g: jnp.sum(x_new[g], axis=0, keepdims=True)
                   / jnp.maximum(cnt[g], 1.0))
        mx = each(lambda g: jnp.max(
            jnp.where(keep[g] > 0.0, x_new[g], _NEG_BIG),
            axis=0, keepdims=True))
        mx = each(lambda g: jnp.where(cnt[g] > 0.0, mx[g], 0.0))
        ro = each(lambda g: jnp.concatenate([avg[g], mx[g]], axis=1))
        return x_new, keep, ro

    a_raw = each(lambda g: a_ref[g])
    a_bf = each(lambda g: a_raw[g].astype(jnp.bfloat16))
    x0 = each(lambda g: x_ref[g])
    m0 = each(lambda g: m_ref[g])
    x1, m1, r1 = stage(a_raw, a_bf, x0, m0,
                       w1_ref, b1_ref, p1_ref, wp1_ref, n // 2)
    x2, m2, r2 = stage(a_raw, a_bf, x1, m1,
                       w2_ref, b2_ref, p2_ref, wp2_ref, n // 4)
    _, _, r3 = stage(a_raw, a_bf, x2, m2,
                     w3_ref, b3_ref, p3_ref, wp3_ref, n // 8)

    out = each(lambda g: jnp.dot(r1[g], wa_ref[...],
                                 preferred_element_type=jnp.float32)
               + ba_ref[...]
               + jnp.dot(r2[g], wb_ref[...],
                         preferred_element_type=jnp.float32)
               + bb_ref[...]
               + r3[g])
    for g in gs:
        o_ref[g] = out[g]


def kernel(gconv1_w, gconv1_b, lupool1_p, lupool1_w,
           gconv2_w, gconv2_b, lupool2_p, lupool2_w,
           gconv3_w, gconv3_b, lupool3_p, lupool3_w,
           cat_a_w, cat_a_b, cat_b_w, cat_b_b,
           a_raw, x_feat, mask, pad_dmap):
    del pad_dmap
    B, N, Fin = x_feat.shape
    E = cat_a_w.shape[1]

    const = lambda shape: pl.BlockSpec(shape, lambda i: (0,) * len(shape))
    weights = [
        gconv1_w, gconv1_b.reshape(1, -1), lupool1_p, lupool1_w,
        gconv2_w, gconv2_b.reshape(1, -1), lupool2_p, lupool2_w,
        gconv3_w, gconv3_b.reshape(1, -1), lupool3_p, lupool3_w,
        cat_a_w, cat_a_b.reshape(1, -1), cat_b_w, cat_b_b.reshape(1, -1),
    ]

    G = 4 if B % 4 == 0 else (2 if B % 2 == 0 else 1)
    out = pl.pallas_call(
        functools.partial(_fused_kernel, n=N),
        out_shape=jax.ShapeDtypeStruct((B, 1, E), jnp.float32),
        grid=(B // G,),
        in_specs=[
            pl.BlockSpec((G, N, N), lambda i: (i, 0, 0)),
            pl.BlockSpec((G, N, Fin), lambda i: (i, 0, 0)),
            pl.BlockSpec((G, N, 1), lambda i: (i, 0, 0)),
        ] + [const(w.shape) for w in weights],
        out_specs=pl.BlockSpec((G, 1, E), lambda i: (i, 0, 0)),
        compiler_params=pltpu.CompilerParams(
            dimension_semantics=("parallel",)),
    )(a_raw, x_feat, mask, *weights)
    return out.reshape(B, E)


# speedup vs baseline: 2.4780x; 1.0719x over previous
import functools

import jax
import jax.numpy as jnp
from jax.experimental import pallas as pl
from jax.experimental.pallas import tpu as pltpu

_NEG_BIG = -1e30


def _outer(u, v):
    return jax.lax.dot_general(u, v, (((1,), (1,)), ((), ())),
                               preferred_element_type=jnp.float32)


def _fused_kernel(a_ref, x_ref, m_ref,
                  w1_ref, b1_ref, p1_ref, wp1_ref,
                  w2_ref, b2_ref, p2_ref, wp2_ref,
                  w3_ref, b3_ref, p3_ref, wp3_ref,
                  wa_ref, ba_ref, wb_ref, bb_ref,
                  o_ref, *, n):
    idx_r = jax.lax.broadcasted_iota(jnp.int32, (n, n), 0)
    idx_c = jax.lax.broadcasted_iota(jnp.int32, (n, n), 1)
    ones = jnp.ones((n, 1), jnp.float32)
    tie = idx_r < idx_c
    npair = a_ref.shape[0]

    gs = range(npair)

    def each(f):
        return [f(g) for g in gs]

    def stage(a_raw, a_bf, x, m, w_ref, b_ref, p_ref, wp_ref, k_keep):
        deg = each(lambda g: m[g] * jnp.dot(
            a_bf[g], m[g].astype(jnp.bfloat16),
            preferred_element_type=jnp.float32))
        dinv = each(lambda g: jnp.where(
            deg[g] > 0.0, jax.lax.rsqrt(jnp.maximum(deg[g], 1e-12)), 0.0))
        dinv_b = each(lambda g: dinv[g].astype(jnp.bfloat16)
                      .astype(jnp.float32))
        dinv_t = each(lambda g: jax.lax.transpose(dinv_b[g], (1, 0)))
        a_norm = each(lambda g: (a_raw[g] * dinv_b[g]) * dinv_t[g])

        xw = each(lambda g: jnp.dot(x[g], w_ref[...],
                                    preferred_element_type=jnp.float32))
        h = each(lambda g: jnp.dot(a_norm[g], xw[g],
                                   preferred_element_type=jnp.float32)
                 + b_ref[...])
        h = each(lambda g: jnp.maximum(h[g], 0.0) * m[g])

        s = each(lambda g: jnp.dot(h[g], p_ref[...],
                                   preferred_element_type=jnp.float32))
        s = each(lambda g: jnp.where(m[g] > 0.0, s[g], _NEG_BIG))

        s_cols = each(lambda g: jax.lax.transpose(
            s[g].astype(jnp.bfloat16).astype(jnp.float32), (1, 0)))
        beats = each(lambda g: jnp.logical_or(
            s[g] > s_cols[g], jnp.logical_and(s[g] == s_cols[g], tie)))
        wins = each(lambda g: jnp.dot(
            beats[g].astype(jnp.bfloat16), ones.astype(jnp.bfloat16),
            preferred_element_type=jnp.float32))
        keep = each(lambda g: jnp.where((n - 1.0) - wins[g] < k_keep,
                                        m[g], 0.0))

        x_new = each(lambda g: jnp.dot(
            h[g] * jnp.tanh(s[g]) * keep[g], wp_ref[...],
            preferred_element_type=jnp.float32) * keep[g])

        cnt = each(lambda g: jnp.sum(keep[g], axis=0, keepdims=True))
        avg = each(lambda g: jnp.sum(x_new[g], axis=0, keepdims=True)
                   / jnp.maximum(cnt[g], 1.0))
        mx = each(lambda g: jnp.max(
            jnp.where(keep[g] > 0.0, x_new[g], _NEG_BIG),
            axis=0, keepdims=True))
        mx = each(lambda g: jnp.where(cnt[g] > 0.0, mx[g], 0.0))
        ro = each(lambda g: jnp.concatenate([avg[g], mx[g]], axis=1))
        return x_new, keep, ro

    a_raw = each(lambda g: a_ref[g])
    a_bf = each(lambda g: a_raw[g].astype(jnp.bfloat16))
    x0 = each(lambda g: x_ref[g])
    m0 = each(lambda g: m_ref[g])
    x1, m1, r1 = stage(a_raw, a_bf, x0, m0,
                       w1_ref, b1_ref, p1_ref, wp1_ref, n // 2)
    x2, m2, r2 = stage(a_raw, a_bf, x1, m1,
                       w2_ref, b2_ref, p2_ref, wp2_ref, n // 4)
    _, _, r3 = stage(a_raw, a_bf, x2, m2,
                     w3_ref, b3_ref, p3_ref, wp3_ref, n // 8)

    out = each(lambda g: jnp.dot(r1[g], wa_ref[...],
                                 preferred_element_type=jnp.float32)
               + ba_ref[...]
               + jnp.dot(r2[g], wb_ref[...],
                         preferred_element_type=jnp.float32)
               + bb_ref[...]
               + r3[g])
    for g in gs:
        o_ref[g] = out[g]


def kernel(gconv1_w, gconv1_b, lupool1_p, lupool1_w,
           gconv2_w, gconv2_b, lupool2_p, lupool2_w,
           gconv3_w, gconv3_b, lupool3_p, lupool3_w,
           cat_a_w, cat_a_b, cat_b_w, cat_b_b,
           a_raw, x_feat, mask, pad_dmap):
    del pad_dmap
    B, N, Fin = x_feat.shape
    E = cat_a_w.shape[1]

    const = lambda shape: pl.BlockSpec(shape, lambda i: (0,) * len(shape))
    weights = [
        gconv1_w, gconv1_b.reshape(1, -1), lupool1_p, lupool1_w,
        gconv2_w, gconv2_b.reshape(1, -1), lupool2_p, lupool2_w,
        gconv3_w, gconv3_b.reshape(1, -1), lupool3_p, lupool3_w,
        cat_a_w, cat_a_b.reshape(1, -1), cat_b_w, cat_b_b.reshape(1, -1),
    ]

    G = 4 if B % 4 == 0 else (2 if B % 2 == 0 else 1)
    out = pl.pallas_call(
        functools.partial(_fused_kernel, n=N),
        out_shape=jax.ShapeDtypeStruct((B, 1, E), jnp.float32),
        grid=(B // G,),
        in_specs=[
            pl.BlockSpec((G, N, N), lambda i: (i, 0, 0)),
            pl.BlockSpec((G, N, Fin), lambda i: (i, 0, 0)),
            pl.BlockSpec((G, N, 1), lambda i: (i, 0, 0)),
        ] + [const(w.shape) for w in weights],
        out_specs=pl.BlockSpec((G, 1, E), lambda i: (i, 0, 0)),
        compiler_params=pltpu.CompilerParams(
            dimension_semantics=("parallel",)),
    )(a_raw, x_feat, mask, *weights)
    return out.reshape(B, E)
